# baseline (speedup 1.0000x reference)
import functools

import jax
import jax.numpy as jnp
from jax import lax
from jax.experimental import pallas as pl
from jax.experimental.pallas import tpu as pltpu
from jax.experimental.pallas import tpu_sc as plsc

_SC_CORES = 2
_SC_SUBCORES = 16
_NW = _SC_CORES * _SC_SUBCORES


def _erf(x):
    a1, a2, a3, a4, a5 = (0.254829592, -0.284496736, 1.421413741,
                          -1.453152027, 1.061405429)
    p = 0.3275911
    ax = jnp.abs(x)
    t = 1.0 / (1.0 + p * ax)
    poly = ((((a5 * t + a4) * t + a3) * t + a2) * t + a1) * t
    y = 1.0 - poly * jnp.exp(-ax * ax)
    return jnp.sign(x) * y


def _gelu(x):
    return 0.5 * x * (1.0 + _erf(x * 0.7071067811865476))


def _layer_norm(x, w, b, eps=1e-5):
    mu = jnp.mean(x, axis=-1, keepdims=True)
    var = jnp.mean((x - mu) ** 2, axis=-1, keepdims=True)
    return (x - mu) / jnp.sqrt(var + eps) * w + b


def _idx_body(pe_rows, ml_ref, idx_s_ref, idx_p_ref, idx_e_ref, mask_ref):
    ml = ml_ref[...]
    Bb, Ss = ml.shape
    mask = ml == 1
    j = lax.broadcasted_iota(jnp.int32, (Bb, Ss), 1)
    a = jnp.where(mask, jnp.int32(-1), j)
    d = 1
    while d < Ss:
        sh = jnp.concatenate(
            [jnp.full((Bb, d), -1, jnp.int32), a[:, : Ss - d]], axis=1)
        a = jnp.maximum(a, sh)
        d *= 2
    last_zero = a
    b = jnp.where(mask, jnp.int32(Ss), j)
    d = 1
    while d < Ss:
        sh = jnp.concatenate(
            [b[:, d:], jnp.full((Bb, d), Ss, jnp.int32)], axis=1)
        b = jnp.minimum(b, sh)
        d *= 2
    next_zero = b
    base = lax.broadcasted_iota(jnp.int32, (Bb, Ss), 0) * Ss
    i_s = jnp.where(last_zero < 0, Ss - 1, last_zero)
    i_e = jnp.minimum(next_zero, Ss - 1)
    i_p = jnp.clip(j - (last_zero + 1), 0, pe_rows - 1)
    idx_s_ref[...] = base + jnp.where(mask, i_s, j)
    idx_e_ref[...] = base + jnp.where(mask, i_e, j)
    idx_p_ref[...] = jnp.where(mask, i_p, 0)
    mask_ref[...] = mask.astype(jnp.float32)


def _idx_call(mask_labels, pe_rows):
    Bb, Ss = mask_labels.shape
    return pl.pallas_call(
        functools.partial(_idx_body, pe_rows),
        out_shape=(
            jax.ShapeDtypeStruct((Bb, Ss), jnp.int32),
            jax.ShapeDtypeStruct((Bb, Ss), jnp.int32),
            jax.ShapeDtypeStruct((Bb, Ss), jnp.int32),
            jax.ShapeDtypeStruct((Bb, Ss), jnp.float32),
        ),
    )(mask_labels)


def _sc_gather_body(hid, pe_tab, i_s, i_p, i_e, o_s, o_p, o_e,
                    idx_v, rows_v, sem):
    wid = lax.axis_index("s") * _SC_CORES + lax.axis_index("c")
    bpw = o_s.shape[0] // _NW
    base = wid * bpw
    for ih, tab, oh in ((i_s, hid, o_s), (i_p, pe_tab, o_p), (i_e, hid, o_e)):
        pltpu.sync_copy(ih.at[pl.ds(base, bpw)], idx_v)
        pltpu.async_copy(tab.at[idx_v], rows_v, sem).wait()
        pltpu.sync_copy(rows_v, oh.at[pl.ds(base, bpw)])


def _sc_gather(hid2, pe_tab, idx_s, idx_p, idx_e):
    rows, d = hid2.shape
    bpw = rows // _NW
    mesh = plsc.VectorSubcoreMesh(core_axis_name="c", subcore_axis_name="s")
    f = pl.kernel(
        _sc_gather_body,
        out_type=(
            jax.ShapeDtypeStruct((rows, d), jnp.float32),
            jax.ShapeDtypeStruct((rows, d), jnp.float32),
            jax.ShapeDtypeStruct((rows, d), jnp.float32),
        ),
        mesh=mesh,
        scratch_types=[
            pltpu.VMEM((bpw,), jnp.int32),
            pltpu.VMEM((bpw, d), jnp.float32),
            pltpu.SemaphoreType.DMA,
        ],
    )
    return f(hid2, pe_tab, idx_s, idx_p, idx_e)


def _proj_body(x_ref, wp_ref, h_ref):
    h_ref[...] = lax.dot_general(
        x_ref[...], wp_ref[...], (((1,), (1,)), ((), ())),
        preferred_element_type=jnp.float32)


def _proj_call(x2, W_proj, bm):
    rows, d = x2.shape
    d3 = W_proj.shape[0]
    return pl.pallas_call(
        _proj_body,
        grid=(rows // bm,),
        in_specs=[
            pl.BlockSpec((bm, d), lambda i: (i, 0)),
            pl.BlockSpec((d3, d), lambda i: (0, 0)),
        ],
        out_specs=pl.BlockSpec((bm, d3), lambda i: (i, 0)),
        out_shape=jax.ShapeDtypeStruct((rows, d3), jnp.float32),
    )(x2, W_proj)


def _ffn_body(h_ref, m_ref, svs_ref, svp_ref, sve_ref, w1_ref, b1_ref,
              l1w_ref, l1b_ref, w2_ref, l2w_ref, l2b_ref, z_ref):
    m = m_ref[...] > 0.0
    dd = svs_ref.shape[1]
    bm = m.shape[0]
    ffn = w1_ref.shape[0]
    acc = jnp.zeros((bm, ffn), jnp.float32)
    svs = (svs_ref, svp_ref, sve_ref)
    for c in range(3):
        h_c = h_ref[:, c * dd:(c + 1) * dd]
        mc = jnp.where(m, svs[c][...], h_c).astype(jnp.bfloat16)
        w1_c = w1_ref[:, c * dd:(c + 1) * dd]
        acc = acc + lax.dot_general(mc, w1_c, (((1,), (1,)), ((), ())),
                                    preferred_element_type=jnp.float32)
    t = _layer_norm(_gelu(acc + b1_ref[...]), l1w_ref[...], l1b_ref[...])
    z = lax.dot_general(t.astype(jnp.bfloat16), w2_ref[...],
                        (((1,), (1,)), ((), ())),
                        preferred_element_type=jnp.float32)
    z = _layer_norm(_gelu(z), l2w_ref[...], l2b_ref[...])
    z_ref[...] = z


def _ffn_call(h2, mask2, sv_s, sv_p, sv_e, W1, b1r, ln1_wr, ln1_br,
              W2, ln2_wr, ln2_br, bm):
    rows, d3 = h2.shape
    d = sv_s.shape[1]
    ffn = W1.shape[0]
    g = rows // bm
    tok = lambda i: (i, 0)
    const = lambda i: (0, 0)
    return pl.pallas_call(
        _ffn_body,
        grid=(g,),
        in_specs=[
            pl.BlockSpec((bm, d3), tok),
            pl.BlockSpec((bm, 1), tok),
            pl.BlockSpec((bm, d), tok),
            pl.BlockSpec((bm, d), tok),
            pl.BlockSpec((bm, d), tok),
            pl.BlockSpec((ffn, d3), const),
            pl.BlockSpec((1, ffn), const),
            pl.BlockSpec((1, ffn), const),
            pl.BlockSpec((1, ffn), const),
            pl.BlockSpec((d, ffn), const),
            pl.BlockSpec((1, d), const),
            pl.BlockSpec((1, d), const),
        ],
        out_specs=pl.BlockSpec((bm, d), tok),
        out_shape=jax.ShapeDtypeStruct((rows, d), jnp.float32),
    )(h2, mask2, sv_s, sv_p, sv_e, W1, b1r, ln1_wr, ln1_br,
      W2, ln2_wr, ln2_br)


def _cls_body(ss, bb, z_ref, wv_ref, bv_ref, out_ref):
    wv = wv_ref[...]
    bv = bv_ref[...]
    for b in range(bb):
        zb = z_ref[pl.ds(b * ss, ss), :]
        out_ref[:, b, :] = lax.dot_general(
            wv, zb, (((1,), (1,)), ((), ())),
            preferred_element_type=jnp.float32) + bv


def _cls_call(z2, W_cls, b_cls_c, vb, bb):
    rows, d = z2.shape
    ss = rows // bb
    v = W_cls.shape[0]
    nv = pl.cdiv(v, vb)
    return pl.pallas_call(
        functools.partial(_cls_body, ss, bb),
        grid=(nv,),
        in_specs=[
            pl.BlockSpec((rows, d), lambda j: (0, 0)),
            pl.BlockSpec((vb, d), lambda j: (j, 0)),
            pl.BlockSpec((vb, 1), lambda j: (j, 0)),
        ],
        out_specs=pl.BlockSpec((vb, bb, ss), lambda j: (j, 0, 0)),
        out_shape=jax.ShapeDtypeStruct((v, bb, ss), jnp.float32),
    )(z2, W_cls, b_cls_c)


def kernel(hidden_states, mask_labels, W_proj, span_pos_emb, W1, b1,
           ln1_w, ln1_b, W2, ln2_w, ln2_b, W_cls, b_cls):
    Bb, Ss, d = hidden_states.shape
    rows = Bb * Ss
    v = W_cls.shape[0]
    pe_rows = span_pos_emb.shape[0]

    idx_s, idx_p, idx_e, mask_f = _idx_call(mask_labels, pe_rows)
    hid2 = hidden_states.reshape(rows, d)
    sv_s, sv_p, sv_e = _sc_gather(
        hid2, span_pos_emb, idx_s.reshape(rows), idx_p.reshape(rows),
        idx_e.reshape(rows))
    h2 = _proj_call(hid2, W_proj, bm=512)
    z = _ffn_call(
        h2, mask_f.reshape(rows, 1), sv_s, sv_p, sv_e,
        W1.astype(jnp.bfloat16),
        b1.reshape(1, -1), ln1_w.reshape(1, -1), ln1_b.reshape(1, -1),
        W2.astype(jnp.bfloat16), ln2_w.reshape(1, -1),
        ln2_b.reshape(1, -1), bm=256)
    logits_t = _cls_call(z.astype(jnp.bfloat16), W_cls.astype(jnp.bfloat16),
                         b_cls.reshape(-1, 1), vb=512, bb=Bb)
    return jnp.transpose(logits_t, (1, 2, 0))


# speedup vs baseline: 1.0577x; 1.0577x over previous
import functools

import jax
import jax.numpy as jnp
from jax import lax
from jax.experimental import pallas as pl
from jax.experimental.pallas import tpu as pltpu
from jax.experimental.pallas import tpu_sc as plsc

_SC_CORES = 2
_SC_SUBCORES = 16
_NW = _SC_CORES * _SC_SUBCORES


def _erf(x):
    a1, a2, a3, a4, a5 = (0.254829592, -0.284496736, 1.421413741,
                          -1.453152027, 1.061405429)
    p = 0.3275911
    ax = jnp.abs(x)
    t = 1.0 / (1.0 + p * ax)
    poly = ((((a5 * t + a4) * t + a3) * t + a2) * t + a1) * t
    y = 1.0 - poly * jnp.exp(-ax * ax)
    return jnp.sign(x) * y


def _gelu(x):
    return 0.5 * x * (1.0 + _erf(x * 0.7071067811865476))


def _layer_norm(x, w, b, eps=1e-5):
    mu = jnp.mean(x, axis=-1, keepdims=True)
    var = jnp.mean((x - mu) ** 2, axis=-1, keepdims=True)
    return (x - mu) / jnp.sqrt(var + eps) * w + b


def _idx_body(pe_rows, ml_ref, idx_s_ref, idx_p_ref, idx_e_ref, mask_ref):
    ml = ml_ref[...]
    Bb, Ss = ml.shape
    mask = ml == 1
    j = lax.broadcasted_iota(jnp.int32, (Bb, Ss), 1)
    a = jnp.where(mask, jnp.int32(-1), j)
    d = 1
    while d < Ss:
        sh = jnp.concatenate(
            [jnp.full((Bb, d), -1, jnp.int32), a[:, : Ss - d]], axis=1)
        a = jnp.maximum(a, sh)
        d *= 2
    last_zero = a
    b = jnp.where(mask, jnp.int32(Ss), j)
    d = 1
    while d < Ss:
        sh = jnp.concatenate(
            [b[:, d:], jnp.full((Bb, d), Ss, jnp.int32)], axis=1)
        b = jnp.minimum(b, sh)
        d *= 2
    next_zero = b
    base = lax.broadcasted_iota(jnp.int32, (Bb, Ss), 0) * Ss
    i_s = jnp.where(last_zero < 0, Ss - 1, last_zero)
    i_e = jnp.minimum(next_zero, Ss - 1)
    i_p = jnp.clip(j - (last_zero + 1), 0, pe_rows - 1)
    idx_s_ref[...] = base + jnp.where(mask, i_s, j)
    idx_e_ref[...] = base + jnp.where(mask, i_e, j)
    idx_p_ref[...] = jnp.where(mask, i_p, 0)
    mask_ref[...] = mask.astype(jnp.float32)


def _idx_call(mask_labels, pe_rows):
    Bb, Ss = mask_labels.shape
    return pl.pallas_call(
        functools.partial(_idx_body, pe_rows),
        out_shape=(
            jax.ShapeDtypeStruct((Bb, Ss), jnp.int32),
            jax.ShapeDtypeStruct((Bb, Ss), jnp.int32),
            jax.ShapeDtypeStruct((Bb, Ss), jnp.int32),
            jax.ShapeDtypeStruct((Bb, Ss), jnp.float32),
        ),
    )(mask_labels)


def _sc_gather_body(hid, pe_tab, i_s, i_p, i_e, o_s, o_p, o_e,
                    idx_v, rows_v, sem):
    wid = lax.axis_index("s") * _SC_CORES + lax.axis_index("c")
    bpw = o_s.shape[0] // _NW
    base = wid * bpw
    for ih, tab, oh in ((i_s, hid, o_s), (i_p, pe_tab, o_p), (i_e, hid, o_e)):
        pltpu.sync_copy(ih.at[pl.ds(base, bpw)], idx_v)
        pltpu.async_copy(tab.at[idx_v], rows_v, sem).wait()
        pltpu.sync_copy(rows_v, oh.at[pl.ds(base, bpw)])


def _sc_gather(hid2, pe_tab, idx_s, idx_p, idx_e):
    rows, d = hid2.shape
    bpw = rows // _NW
    mesh = plsc.VectorSubcoreMesh(core_axis_name="c", subcore_axis_name="s")
    f = pl.kernel(
        _sc_gather_body,
        out_type=(
            jax.ShapeDtypeStruct((rows, d), jnp.float32),
            jax.ShapeDtypeStruct((rows, d), jnp.float32),
            jax.ShapeDtypeStruct((rows, d), jnp.float32),
        ),
        mesh=mesh,
        scratch_types=[
            pltpu.VMEM((bpw,), jnp.int32),
            pltpu.VMEM((bpw, d), jnp.float32),
            pltpu.SemaphoreType.DMA,
        ],
    )
    return f(hid2, pe_tab, idx_s, idx_p, idx_e)


def _proj_body(x_ref, wp_ref, h_ref):
    h_ref[...] = lax.dot_general(
        x_ref[...], wp_ref[...], (((1,), (1,)), ((), ())),
        preferred_element_type=jnp.float32)


def _proj_call(x2, W_proj, bm):
    rows, d = x2.shape
    d3 = W_proj.shape[0]
    return pl.pallas_call(
        _proj_body,
        grid=(rows // bm,),
        in_specs=[
            pl.BlockSpec((bm, d), lambda i: (i, 0)),
            pl.BlockSpec((d3, d), lambda i: (0, 0)),
        ],
        out_specs=pl.BlockSpec((bm, d3), lambda i: (i, 0)),
        out_shape=jax.ShapeDtypeStruct((rows, d3), jnp.float32),
    )(x2, W_proj)


def _ffn_body(h_ref, m_ref, svs_ref, svp_ref, sve_ref, w1_ref, b1_ref,
              l1w_ref, l1b_ref, w2_ref, l2w_ref, l2b_ref, z_ref):
    m = m_ref[...] > 0.0
    dd = svs_ref.shape[1]
    bm = m.shape[0]
    ffn = w1_ref.shape[0]
    acc = jnp.zeros((bm, ffn), jnp.float32)
    svs = (svs_ref, svp_ref, sve_ref)
    for c in range(3):
        h_c = h_ref[:, c * dd:(c + 1) * dd]
        mc = jnp.where(m, svs[c][...], h_c)
        w1_c = w1_ref[:, c * dd:(c + 1) * dd]
        acc = acc + lax.dot_general(mc, w1_c, (((1,), (1,)), ((), ())),
                                    preferred_element_type=jnp.float32)
    t = _layer_norm(_gelu(acc + b1_ref[...]), l1w_ref[...], l1b_ref[...])
    z = lax.dot_general(t, w2_ref[...], (((1,), (1,)), ((), ())),
                        preferred_element_type=jnp.float32)
    z = _layer_norm(_gelu(z), l2w_ref[...], l2b_ref[...])
    z_ref[...] = z


def _ffn_call(h2, mask2, sv_s, sv_p, sv_e, W1, b1r, ln1_wr, ln1_br,
              W2, ln2_wr, ln2_br, bm):
    rows, d3 = h2.shape
    d = sv_s.shape[1]
    ffn = W1.shape[0]
    g = rows // bm
    tok = lambda i: (i, 0)
    const = lambda i: (0, 0)
    return pl.pallas_call(
        _ffn_body,
        grid=(g,),
        in_specs=[
            pl.BlockSpec((bm, d3), tok),
            pl.BlockSpec((bm, 1), tok),
            pl.BlockSpec((bm, d), tok),
            pl.BlockSpec((bm, d), tok),
            pl.BlockSpec((bm, d), tok),
            pl.BlockSpec((ffn, d3), const),
            pl.BlockSpec((1, ffn), const),
            pl.BlockSpec((1, ffn), const),
            pl.BlockSpec((1, ffn), const),
            pl.BlockSpec((d, ffn), const),
            pl.BlockSpec((1, d), const),
            pl.BlockSpec((1, d), const),
        ],
        out_specs=pl.BlockSpec((bm, d), tok),
        out_shape=jax.ShapeDtypeStruct((rows, d), jnp.float32),
    )(h2, mask2, sv_s, sv_p, sv_e, W1, b1r, ln1_wr, ln1_br,
      W2, ln2_wr, ln2_br)


def _cls_body(ss, bb, z_ref, wv_ref, bv_ref, out_ref):
    wv = wv_ref[...]
    bv = bv_ref[...]
    for b in range(bb):
        zb = z_ref[pl.ds(b * ss, ss), :]
        out_ref[:, b, :] = lax.dot_general(
            wv, zb, (((1,), (1,)), ((), ())),
            preferred_element_type=jnp.float32) + bv


def _cls_call(z2, W_cls, b_cls_c, vb, bb):
    rows, d = z2.shape
    ss = rows // bb
    v = W_cls.shape[0]
    nv = pl.cdiv(v, vb)
    return pl.pallas_call(
        functools.partial(_cls_body, ss, bb),
        grid=(nv,),
        in_specs=[
            pl.BlockSpec((rows, d), lambda j: (0, 0)),
            pl.BlockSpec((vb, d), lambda j: (j, 0)),
            pl.BlockSpec((vb, 1), lambda j: (j, 0)),
        ],
        out_specs=pl.BlockSpec((vb, bb, ss), lambda j: (j, 0, 0)),
        out_shape=jax.ShapeDtypeStruct((v, bb, ss), jnp.float32),
    )(z2, W_cls, b_cls_c)


def kernel(hidden_states, mask_labels, W_proj, span_pos_emb, W1, b1,
           ln1_w, ln1_b, W2, ln2_w, ln2_b, W_cls, b_cls):
    Bb, Ss, d = hidden_states.shape
    rows = Bb * Ss
    v = W_cls.shape[0]
    pe_rows = span_pos_emb.shape[0]

    idx_s, idx_p, idx_e, mask_f = _idx_call(mask_labels, pe_rows)
    hid2 = hidden_states.reshape(rows, d)
    sv_s, sv_p, sv_e = _sc_gather(
        hid2, span_pos_emb, idx_s.reshape(rows), idx_p.reshape(rows),
        idx_e.reshape(rows))
    h2 = _proj_call(hid2, W_proj, bm=512)
    z = _ffn_call(
        h2, mask_f.reshape(rows, 1), sv_s, sv_p, sv_e, W1,
        b1.reshape(1, -1), ln1_w.reshape(1, -1), ln1_b.reshape(1, -1),
        W2, ln2_w.reshape(1, -1), ln2_b.reshape(1, -1), bm=256)
    logits_t = _cls_call(z, W_cls, b_cls.reshape(-1, 1), vb=512, bb=Bb)
    return jnp.transpose(logits_t, (1, 2, 0))


# speedup vs baseline: 1.3838x; 1.3083x over previous
import functools

import jax
import jax.numpy as jnp
from jax import lax
from jax.experimental import pallas as pl
from jax.experimental.pallas import tpu as pltpu
from jax.experimental.pallas import tpu_sc as plsc

_SC_CORES = 2
_SC_SUBCORES = 16
_NW = _SC_CORES * _SC_SUBCORES


def _erf(x):
    a1, a2, a3, a4, a5 = (0.254829592, -0.284496736, 1.421413741,
                          -1.453152027, 1.061405429)
    p = 0.3275911
    ax = jnp.abs(x)
    t = 1.0 / (1.0 + p * ax)
    poly = ((((a5 * t + a4) * t + a3) * t + a2) * t + a1) * t
    y = 1.0 - poly * jnp.exp(-ax * ax)
    return jnp.sign(x) * y


def _gelu(x):
    return 0.5 * x * (1.0 + _erf(x * 0.7071067811865476))


def _layer_norm(x, w, b, eps=1e-5):
    mu = jnp.mean(x, axis=-1, keepdims=True)
    var = jnp.mean((x - mu) ** 2, axis=-1, keepdims=True)
    return (x - mu) / jnp.sqrt(var + eps) * w + b


def _idx_body(pe_rows, ml_ref, idx_s_ref, idx_p_ref, idx_e_ref, mask_ref):
    ml = ml_ref[...]
    Bb, Ss = ml.shape
    mask = ml == 1
    j = lax.broadcasted_iota(jnp.int32, (Bb, Ss), 1)
    a = jnp.where(mask, jnp.int32(-1), j)
    d = 1
    while d < Ss:
        sh = jnp.concatenate(
            [jnp.full((Bb, d), -1, jnp.int32), a[:, : Ss - d]], axis=1)
        a = jnp.maximum(a, sh)
        d *= 2
    last_zero = a
    b = jnp.where(mask, jnp.int32(Ss), j)
    d = 1
    while d < Ss:
        sh = jnp.concatenate(
            [b[:, d:], jnp.full((Bb, d), Ss, jnp.int32)], axis=1)
        b = jnp.minimum(b, sh)
        d *= 2
    next_zero = b
    base = lax.broadcasted_iota(jnp.int32, (Bb, Ss), 0) * Ss
    i_s = jnp.where(last_zero < 0, Ss - 1, last_zero)
    i_e = jnp.minimum(next_zero, Ss - 1)
    i_p = jnp.clip(j - (last_zero + 1), 0, pe_rows - 1)
    idx_s_ref[...] = base + jnp.where(mask, i_s, j)
    idx_e_ref[...] = base + jnp.where(mask, i_e, j)
    idx_p_ref[...] = jnp.where(mask, i_p, 0)
    mask_ref[...] = mask.astype(jnp.float32)


def _idx_call(mask_labels, pe_rows):
    Bb, Ss = mask_labels.shape
    return pl.pallas_call(
        functools.partial(_idx_body, pe_rows),
        out_shape=(
            jax.ShapeDtypeStruct((Bb, Ss), jnp.int32),
            jax.ShapeDtypeStruct((Bb, Ss), jnp.int32),
            jax.ShapeDtypeStruct((Bb, Ss), jnp.int32),
            jax.ShapeDtypeStruct((Bb, Ss), jnp.float32),
        ),
    )(mask_labels)


def _sc_gather_body(hid, i_s, i_e, o_s, o_e,
                    idx_sv, idx_ev, buf_a, buf_b, gsa, gsb, osa, osb):
    wid = lax.axis_index("s") * _SC_CORES + lax.axis_index("c")
    bpw = o_s.shape[0] // _NW
    half = bpw // 2
    base = wid * bpw
    pltpu.sync_copy(i_s.at[pl.ds(base, bpw)], idx_sv)
    pltpu.sync_copy(i_e.at[pl.ds(base, bpw)], idx_ev)
    chunks = (
        (idx_sv, 0, o_s, base),
        (idx_sv, half, o_s, base + half),
        (idx_ev, 0, o_e, base),
        (idx_ev, half, o_e, base + half),
    )
    bufs = (buf_a, buf_b)
    gsems = (gsa, gsb)
    osems = (osa, osb)
    gd = [None, None]
    od = [None, None]
    for i, (iv, ioff, oh, ooff) in enumerate(chunks):
        b = i % 2
        if od[b] is not None:
            od[b].wait()
        gd[b] = pltpu.async_copy(
            hid.at[iv.at[pl.ds(ioff, half)]], bufs[b], gsems[b])
        if i >= 1:
            pb = (i - 1) % 2
            gd[pb].wait()
            _, poff, poh, pooff = chunks[i - 1]
            od[pb] = pltpu.async_copy(
                bufs[pb], poh.at[pl.ds(pooff, half)], osems[pb])
    last = len(chunks) - 1
    b = last % 2
    gd[b].wait()
    _, _, oh, ooff = chunks[last]
    od[b] = pltpu.async_copy(bufs[b], oh.at[pl.ds(ooff, half)], osems[b])
    od[(last - 1) % 2].wait()
    od[b].wait()


def _sc_gather(hid2, idx_s, idx_e):
    rows, d = hid2.shape
    bpw = rows // _NW
    mesh = plsc.VectorSubcoreMesh(core_axis_name="c", subcore_axis_name="s")
    f = pl.kernel(
        _sc_gather_body,
        out_type=(
            jax.ShapeDtypeStruct((rows, d), jnp.float32),
            jax.ShapeDtypeStruct((rows, d), jnp.float32),
        ),
        mesh=mesh,
        scratch_types=[
            pltpu.VMEM((bpw,), jnp.int32),
            pltpu.VMEM((bpw,), jnp.int32),
            pltpu.VMEM((bpw // 2, d), jnp.float32),
            pltpu.VMEM((bpw // 2, d), jnp.float32),
            pltpu.SemaphoreType.DMA,
            pltpu.SemaphoreType.DMA,
            pltpu.SemaphoreType.DMA,
            pltpu.SemaphoreType.DMA,
        ],
    )
    return f(hid2, idx_s, idx_e)


def _proj_body(x_ref, wp_ref, h_ref):
    h_ref[...] = lax.dot_general(
        x_ref[...], wp_ref[...], (((1,), (1,)), ((), ())),
        preferred_element_type=jnp.float32)


def _proj_call(x2, W_proj, bm):
    rows, d = x2.shape
    d3 = W_proj.shape[0]
    return pl.pallas_call(
        _proj_body,
        grid=(rows // bm,),
        in_specs=[
            pl.BlockSpec((bm, d), lambda i: (i, 0)),
            pl.BlockSpec((d3, d), lambda i: (0, 0)),
        ],
        out_specs=pl.BlockSpec((bm, d3), lambda i: (i, 0)),
        out_shape=jax.ShapeDtypeStruct((rows, d3), jnp.float32),
    )(x2, W_proj)


def _ffn_body(h_ref, m_ref, svs_ref, ip_ref, pe_ref, sve_ref, w1_ref, b1_ref,
              l1w_ref, l1b_ref, w2_ref, l2w_ref, l2b_ref, z_ref):
    m = m_ref[...] > 0.0
    dd = svs_ref.shape[1]
    bm = m.shape[0]
    ffn = w1_ref.shape[0]
    pe_rows = pe_ref.shape[0]
    oh = (ip_ref[...] == lax.broadcasted_iota(
        jnp.int32, (bm, pe_rows), 1)).astype(jnp.float32)
    pe = lax.dot_general(oh, pe_ref[...], (((1,), (0,)), ((), ())),
                         preferred_element_type=jnp.float32)
    acc = jnp.zeros((bm, ffn), jnp.float32)
    svs = (svs_ref, pe, sve_ref)
    for c in range(3):
        h_c = h_ref[:, c * dd:(c + 1) * dd]
        sv_c = svs[c] if c == 1 else svs[c][...]
        mc = jnp.where(m, sv_c, h_c)
        w1_c = w1_ref[:, c * dd:(c + 1) * dd]
        acc = acc + lax.dot_general(mc, w1_c, (((1,), (1,)), ((), ())),
                                    preferred_element_type=jnp.float32)
    t = _layer_norm(_gelu(acc + b1_ref[...]), l1w_ref[...], l1b_ref[...])
    z = lax.dot_general(t, w2_ref[...], (((1,), (1,)), ((), ())),
                        preferred_element_type=jnp.float32)
    z = _layer_norm(_gelu(z), l2w_ref[...], l2b_ref[...])
    z_ref[...] = z


def _ffn_call(h2, mask2, sv_s, idx_p2, pe_tab, sv_e, W1, b1r, ln1_wr,
              ln1_br, W2, ln2_wr, ln2_br, bm):
    rows, d3 = h2.shape
    d = sv_s.shape[1]
    pe_rows = pe_tab.shape[0]
    ffn = W1.shape[0]
    g = rows // bm
    tok = lambda i: (i, 0)
    const = lambda i: (0, 0)
    return pl.pallas_call(
        _ffn_body,
        grid=(g,),
        in_specs=[
            pl.BlockSpec((bm, d3), tok),
            pl.BlockSpec((bm, 1), tok),
            pl.BlockSpec((bm, d), tok),
            pl.BlockSpec((bm, 1), tok),
            pl.BlockSpec((pe_rows, d), const),
            pl.BlockSpec((bm, d), tok),
            pl.BlockSpec((ffn, d3), const),
            pl.BlockSpec((1, ffn), const),
            pl.BlockSpec((1, ffn), const),
            pl.BlockSpec((1, ffn), const),
            pl.BlockSpec((d, ffn), const),
            pl.BlockSpec((1, d), const),
            pl.BlockSpec((1, d), const),
        ],
        out_specs=pl.BlockSpec((bm, d), tok),
        out_shape=jax.ShapeDtypeStruct((rows, d), jnp.float32),
    )(h2, mask2, sv_s, idx_p2, pe_tab, sv_e, W1, b1r, ln1_wr, ln1_br,
      W2, ln2_wr, ln2_br)


def _cls_body(ss, bb, z_ref, wv_ref, bv_ref, out_ref):
    wv = wv_ref[...]
    bv = bv_ref[...]
    for b in range(bb):
        zb = z_ref[pl.ds(b * ss, ss), :]
        out_ref[:, b, :] = lax.dot_general(
            wv, zb, (((1,), (1,)), ((), ())),
            preferred_element_type=jnp.float32) + bv


def _cls_call(z2, W_cls, b_cls_c, vb, bb):
    rows, d = z2.shape
    ss = rows // bb
    v = W_cls.shape[0]
    nv = pl.cdiv(v, vb)
    return pl.pallas_call(
        functools.partial(_cls_body, ss, bb),
        grid=(nv,),
        in_specs=[
            pl.BlockSpec((rows, d), lambda j: (0, 0)),
            pl.BlockSpec((vb, d), lambda j: (j, 0)),
            pl.BlockSpec((vb, 1), lambda j: (j, 0)),
        ],
        out_specs=pl.BlockSpec((vb, bb, ss), lambda j: (j, 0, 0)),
        out_shape=jax.ShapeDtypeStruct((v, bb, ss), jnp.float32),
    )(z2, W_cls, b_cls_c)


def kernel(hidden_states, mask_labels, W_proj, span_pos_emb, W1, b1,
           ln1_w, ln1_b, W2, ln2_w, ln2_b, W_cls, b_cls):
    Bb, Ss, d = hidden_states.shape
    rows = Bb * Ss
    v = W_cls.shape[0]
    pe_rows = span_pos_emb.shape[0]

    idx_s, idx_p, idx_e, mask_f = _idx_call(mask_labels, pe_rows)
    hid2 = hidden_states.reshape(rows, d)
    sv_s, sv_e = _sc_gather(hid2, idx_s.reshape(rows), idx_e.reshape(rows))
    h2 = _proj_call(hid2, W_proj, bm=512)
    z = _ffn_call(
        h2, mask_f.reshape(rows, 1), sv_s, idx_p.reshape(rows, 1),
        span_pos_emb, sv_e, W1,
        b1.reshape(1, -1), ln1_w.reshape(1, -1), ln1_b.reshape(1, -1),
        W2, ln2_w.reshape(1, -1), ln2_b.reshape(1, -1), bm=256)
    logits_t = _cls_call(z, W_cls, b_cls.reshape(-1, 1), vb=512, bb=Bb)
    return jnp.transpose(logits_t, (1, 2, 0))


# speedup vs baseline: 1.3929x; 1.0066x over previous
import functools

import jax
import jax.numpy as jnp
from jax import lax
from jax.experimental import pallas as pl
from jax.experimental.pallas import tpu as pltpu
from jax.experimental.pallas import tpu_sc as plsc

_SC_CORES = 2
_SC_SUBCORES = 16
_NW = _SC_CORES * _SC_SUBCORES


def _erf(x):
    a1, a2, a3, a4, a5 = (0.254829592, -0.284496736, 1.421413741,
                          -1.453152027, 1.061405429)
    p = 0.3275911
    ax = jnp.abs(x)
    t = 1.0 / (1.0 + p * ax)
    poly = ((((a5 * t + a4) * t + a3) * t + a2) * t + a1) * t
    y = 1.0 - poly * jnp.exp(-ax * ax)
    return jnp.sign(x) * y


def _gelu(x):
    return 0.5 * x * (1.0 + _erf(x * 0.7071067811865476))


def _layer_norm(x, w, b, eps=1e-5):
    mu = jnp.mean(x, axis=-1, keepdims=True)
    var = jnp.mean((x - mu) ** 2, axis=-1, keepdims=True)
    return (x - mu) / jnp.sqrt(var + eps) * w + b


def _idx_body(pe_rows, ml_ref, idx_s_ref, idx_p_ref, idx_e_ref, mask_ref):
    ml = ml_ref[...]
    Bb, Ss = ml.shape
    mask = ml == 1
    j = lax.broadcasted_iota(jnp.int32, (Bb, Ss), 1)
    a = jnp.where(mask, jnp.int32(-1), j)
    d = 1
    while d < Ss:
        sh = jnp.concatenate(
            [jnp.full((Bb, d), -1, jnp.int32), a[:, : Ss - d]], axis=1)
        a = jnp.maximum(a, sh)
        d *= 2
    last_zero = a
    b = jnp.where(mask, jnp.int32(Ss), j)
    d = 1
    while d < Ss:
        sh = jnp.concatenate(
            [b[:, d:], jnp.full((Bb, d), Ss, jnp.int32)], axis=1)
        b = jnp.minimum(b, sh)
        d *= 2
    next_zero = b
    base = lax.broadcasted_iota(jnp.int32, (Bb, Ss), 0) * Ss
    i_s = jnp.where(last_zero < 0, Ss - 1, last_zero)
    i_e = jnp.minimum(next_zero, Ss - 1)
    i_p = jnp.clip(j - (last_zero + 1), 0, pe_rows - 1)
    idx_s_ref[...] = base + jnp.where(mask, i_s, j)
    idx_e_ref[...] = base + jnp.where(mask, i_e, j)
    idx_p_ref[...] = jnp.where(mask, i_p, 0)
    mask_ref[...] = mask.astype(jnp.float32)


def _idx_call(mask_labels, pe_rows):
    Bb, Ss = mask_labels.shape
    return pl.pallas_call(
        functools.partial(_idx_body, pe_rows),
        out_shape=(
            jax.ShapeDtypeStruct((Bb, Ss), jnp.int32),
            jax.ShapeDtypeStruct((Bb, Ss), jnp.int32),
            jax.ShapeDtypeStruct((Bb, Ss), jnp.int32),
            jax.ShapeDtypeStruct((Bb, Ss), jnp.float32),
        ),
    )(mask_labels)


def _sc_gather_body(hid, i_s, i_e, o_s, o_e,
                    idx_sv, idx_ev, buf_a, buf_b, gsa, gsb, osa, osb):
    wid = lax.axis_index("s") * _SC_CORES + lax.axis_index("c")
    bpw = o_s.shape[0] // _NW
    half = bpw // 2
    base = wid * bpw
    pltpu.sync_copy(i_s.at[pl.ds(base, bpw)], idx_sv)
    pltpu.sync_copy(i_e.at[pl.ds(base, bpw)], idx_ev)
    chunks = (
        (idx_sv, 0, o_s, base),
        (idx_sv, half, o_s, base + half),
        (idx_ev, 0, o_e, base),
        (idx_ev, half, o_e, base + half),
    )
    bufs = (buf_a, buf_b)
    gsems = (gsa, gsb)
    osems = (osa, osb)
    gd = [None, None]
    od = [None, None]
    for i, (iv, ioff, oh, ooff) in enumerate(chunks):
        b = i % 2
        if od[b] is not None:
            od[b].wait()
        gd[b] = pltpu.async_copy(
            hid.at[iv.at[pl.ds(ioff, half)]], bufs[b], gsems[b])
        if i >= 1:
            pb = (i - 1) % 2
            gd[pb].wait()
            _, poff, poh, pooff = chunks[i - 1]
            od[pb] = pltpu.async_copy(
                bufs[pb], poh.at[pl.ds(pooff, half)], osems[pb])
    last = len(chunks) - 1
    b = last % 2
    gd[b].wait()
    _, _, oh, ooff = chunks[last]
    od[b] = pltpu.async_copy(bufs[b], oh.at[pl.ds(ooff, half)], osems[b])
    od[(last - 1) % 2].wait()
    od[b].wait()


def _sc_gather(hid2, idx_s, idx_e):
    rows, d = hid2.shape
    bpw = rows // _NW
    mesh = plsc.VectorSubcoreMesh(core_axis_name="c", subcore_axis_name="s")
    f = pl.kernel(
        _sc_gather_body,
        out_type=(
            jax.ShapeDtypeStruct((rows, d), jnp.float32),
            jax.ShapeDtypeStruct((rows, d), jnp.float32),
        ),
        mesh=mesh,
        scratch_types=[
            pltpu.VMEM((bpw,), jnp.int32),
            pltpu.VMEM((bpw,), jnp.int32),
            pltpu.VMEM((bpw // 2, d), jnp.float32),
            pltpu.VMEM((bpw // 2, d), jnp.float32),
            pltpu.SemaphoreType.DMA,
            pltpu.SemaphoreType.DMA,
            pltpu.SemaphoreType.DMA,
            pltpu.SemaphoreType.DMA,
        ],
    )
    return f(hid2, idx_s, idx_e)


def _proj_body(x_ref, wp_ref, h_ref):
    h_ref[...] = lax.dot_general(
        x_ref[...], wp_ref[...], (((1,), (1,)), ((), ())),
        preferred_element_type=jnp.float32)


def _proj_call(x2, W_proj, bm):
    rows, d = x2.shape
    d3 = W_proj.shape[0]
    return pl.pallas_call(
        _proj_body,
        grid=(rows // bm,),
        in_specs=[
            pl.BlockSpec((bm, d), lambda i: (i, 0)),
            pl.BlockSpec((d3, d), lambda i: (0, 0)),
        ],
        out_specs=pl.BlockSpec((bm, d3), lambda i: (i, 0)),
        out_shape=jax.ShapeDtypeStruct((rows, d3), jnp.float32),
    )(x2, W_proj)


def _ffn_body(x_ref, m_ref, svs_ref, ip_ref, pe_ref, sve_ref, wp_ref,
              w1_ref, b1_ref, l1w_ref, l1b_ref, w2_ref, l2w_ref, l2b_ref,
              z_ref):
    x = x_ref[...]
    m = m_ref[...] > 0.0
    dd = x.shape[1]
    bm = x.shape[0]
    ffn = w1_ref.shape[0]
    pe_rows = pe_ref.shape[0]
    oh = (ip_ref[...] == lax.broadcasted_iota(
        jnp.int32, (bm, pe_rows), 1)).astype(jnp.float32)
    pe = lax.dot_general(oh, pe_ref[...], (((1,), (0,)), ((), ())),
                         preferred_element_type=jnp.float32)
    acc = jnp.zeros((bm, ffn), jnp.float32)
    svs = (svs_ref, pe, sve_ref)
    for c in range(3):
        wp_c = wp_ref[c * dd:(c + 1) * dd, :]
        h_c = lax.dot_general(x, wp_c, (((1,), (1,)), ((), ())),
                              preferred_element_type=jnp.float32)
        sv_c = svs[c] if c == 1 else svs[c][...]
        mc = jnp.where(m, sv_c, h_c)
        w1_c = w1_ref[:, c * dd:(c + 1) * dd]
        acc = acc + lax.dot_general(mc, w1_c, (((1,), (1,)), ((), ())),
                                    preferred_element_type=jnp.float32)
    t = _layer_norm(_gelu(acc + b1_ref[...]), l1w_ref[...], l1b_ref[...])
    z = lax.dot_general(t, w2_ref[...], (((1,), (1,)), ((), ())),
                        preferred_element_type=jnp.float32)
    z = _layer_norm(_gelu(z), l2w_ref[...], l2b_ref[...])
    z_ref[...] = z


def _ffn_call(x2, mask2, sv_s, idx_p2, pe_tab, sv_e, W_proj, W1, b1r,
              ln1_wr, ln1_br, W2, ln2_wr, ln2_br, bm):
    rows, d = x2.shape
    d3 = W_proj.shape[0]
    pe_rows = pe_tab.shape[0]
    ffn = W1.shape[0]
    g = rows // bm
    tok = lambda i: (i, 0)
    const = lambda i: (0, 0)
    return pl.pallas_call(
        _ffn_body,
        grid=(g,),
        in_specs=[
            pl.BlockSpec((bm, d), tok),
            pl.BlockSpec((bm, 1), tok),
            pl.BlockSpec((bm, d), tok),
            pl.BlockSpec((bm, 1), tok),
            pl.BlockSpec((pe_rows, d), const),
            pl.BlockSpec((bm, d), tok),
            pl.BlockSpec((d3, d), const),
            pl.BlockSpec((ffn, d3), const),
            pl.BlockSpec((1, ffn), const),
            pl.BlockSpec((1, ffn), const),
            pl.BlockSpec((1, ffn), const),
            pl.BlockSpec((d, ffn), const),
            pl.BlockSpec((1, d), const),
            pl.BlockSpec((1, d), const),
        ],
        out_specs=pl.BlockSpec((bm, d), tok),
        out_shape=jax.ShapeDtypeStruct((rows, d), jnp.float32),
    )(x2, mask2, sv_s, idx_p2, pe_tab, sv_e, W_proj, W1, b1r, ln1_wr,
      ln1_br, W2, ln2_wr, ln2_br)


def _cls_body(ss, bb, z_ref, wv_ref, bv_ref, out_ref):
    wv = wv_ref[...]
    bv = jnp.transpose(bv_ref[...], (1, 0))
    for b in range(bb):
        zb = z_ref[pl.ds(b * ss, ss), :]
        out_ref[:, b, :] = lax.dot_general(
            wv, zb, (((1,), (1,)), ((), ())),
            preferred_element_type=jnp.float32) + bv


def _cls_call(z2, W_cls, b_cls_c, vb, bb):
    rows, d = z2.shape
    ss = rows // bb
    v = W_cls.shape[0]
    nv = pl.cdiv(v, vb)
    return pl.pallas_call(
        functools.partial(_cls_body, ss, bb),
        grid=(nv,),
        in_specs=[
            pl.BlockSpec((rows, d), lambda j: (0, 0)),
            pl.BlockSpec((vb, d), lambda j: (j, 0)),
            pl.BlockSpec((1, vb), lambda j: (0, j)),
        ],
        out_specs=pl.BlockSpec((vb, bb, ss), lambda j: (j, 0, 0)),
        out_shape=jax.ShapeDtypeStruct((v, bb, ss), jnp.float32),
    )(z2, W_cls, b_cls_c)


def kernel(hidden_states, mask_labels, W_proj, span_pos_emb, W1, b1,
           ln1_w, ln1_b, W2, ln2_w, ln2_b, W_cls, b_cls):
    Bb, Ss, d = hidden_states.shape
    rows = Bb * Ss
    v = W_cls.shape[0]
    pe_rows = span_pos_emb.shape[0]

    idx_s, idx_p, idx_e, mask_f = _idx_call(mask_labels, pe_rows)
    hid2 = hidden_states.reshape(rows, d)
    sv_s, sv_e = _sc_gather(hid2, idx_s.reshape(rows), idx_e.reshape(rows))
    z = _ffn_call(
        hid2, mask_f.reshape(rows, 1), sv_s, idx_p.reshape(rows, 1),
        span_pos_emb, sv_e, W_proj, W1,
        b1.reshape(1, -1), ln1_w.reshape(1, -1), ln1_b.reshape(1, -1),
        W2, ln2_w.reshape(1, -1), ln2_b.reshape(1, -1), bm=256)
    logits_t = _cls_call(z, W_cls, b_cls.reshape(1, -1), vb=512, bb=Bb)
    return jnp.transpose(logits_t, (1, 2, 0))


# speedup vs baseline: 1.4563x; 1.0455x over previous
import functools

import jax
import jax.numpy as jnp
from jax import lax
from jax.experimental import pallas as pl
from jax.experimental.pallas import tpu as pltpu
from jax.experimental.pallas import tpu_sc as plsc

_SC_CORES = 2
_SC_SUBCORES = 16
_NW = _SC_CORES * _SC_SUBCORES


def _erf(x):
    a1, a2, a3, a4, a5 = (0.254829592, -0.284496736, 1.421413741,
                          -1.453152027, 1.061405429)
    p = 0.3275911
    ax = jnp.abs(x)
    t = 1.0 / (1.0 + p * ax)
    poly = ((((a5 * t + a4) * t + a3) * t + a2) * t + a1) * t
    y = 1.0 - poly * jnp.exp(-ax * ax)
    return jnp.sign(x) * y


def _gelu(x):
    return 0.5 * x * (1.0 + _erf(x * 0.7071067811865476))


def _layer_norm(x, w, b, eps=1e-5):
    mu = jnp.mean(x, axis=-1, keepdims=True)
    var = jnp.mean((x - mu) ** 2, axis=-1, keepdims=True)
    return (x - mu) / jnp.sqrt(var + eps) * w + b


def _idx_body(pe_rows, ml_ref, idx_s_ref, idx_p_ref, idx_e_ref, mask_ref):
    ml = ml_ref[...]
    Bb, Ss = ml.shape
    mask = ml == 1
    j = lax.broadcasted_iota(jnp.int32, (Bb, Ss), 1)
    a = jnp.where(mask, jnp.int32(-1), j)
    d = 1
    while d < Ss:
        sh = jnp.concatenate(
            [jnp.full((Bb, d), -1, jnp.int32), a[:, : Ss - d]], axis=1)
        a = jnp.maximum(a, sh)
        d *= 2
    last_zero = a
    b = jnp.where(mask, jnp.int32(Ss), j)
    d = 1
    while d < Ss:
        sh = jnp.concatenate(
            [b[:, d:], jnp.full((Bb, d), Ss, jnp.int32)], axis=1)
        b = jnp.minimum(b, sh)
        d *= 2
    next_zero = b
    base = lax.broadcasted_iota(jnp.int32, (Bb, Ss), 0) * Ss
    i_s = jnp.where(last_zero < 0, Ss - 1, last_zero)
    i_e = jnp.minimum(next_zero, Ss - 1)
    i_p = jnp.clip(j - (last_zero + 1), 0, pe_rows - 1)
    idx_s_ref[...] = base + jnp.where(mask, i_s, j)
    idx_e_ref[...] = base + jnp.where(mask, i_e, j)
    idx_p_ref[...] = jnp.where(mask, i_p, 0)
    mask_ref[...] = mask.astype(jnp.float32)


def _idx_call(mask_labels, pe_rows):
    Bb, Ss = mask_labels.shape
    return pl.pallas_call(
        functools.partial(_idx_body, pe_rows),
        out_shape=(
            jax.ShapeDtypeStruct((Bb, Ss), jnp.int32),
            jax.ShapeDtypeStruct((Bb, Ss), jnp.int32),
            jax.ShapeDtypeStruct((Bb, Ss), jnp.int32),
            jax.ShapeDtypeStruct((Bb, Ss), jnp.float32),
        ),
    )(mask_labels)


def _sc_gather_body(hid, i_s, i_e, o_s, o_e,
                    idx_sv, idx_ev, buf_a, buf_b, gsa, gsb, osa, osb):
    wid = lax.axis_index("s") * _SC_CORES + lax.axis_index("c")
    bpw = o_s.shape[0] // _NW
    half = bpw // 2
    base = wid * bpw
    pltpu.sync_copy(i_s.at[pl.ds(base, bpw)], idx_sv)
    pltpu.sync_copy(i_e.at[pl.ds(base, bpw)], idx_ev)
    chunks = (
        (idx_sv, 0, o_s, base),
        (idx_sv, half, o_s, base + half),
        (idx_ev, 0, o_e, base),
        (idx_ev, half, o_e, base + half),
    )
    bufs = (buf_a, buf_b)
    gsems = (gsa, gsb)
    osems = (osa, osb)
    gd = [None, None]
    od = [None, None]
    for i, (iv, ioff, oh, ooff) in enumerate(chunks):
        b = i % 2
        if od[b] is not None:
            od[b].wait()
        gd[b] = pltpu.async_copy(
            hid.at[iv.at[pl.ds(ioff, half)]], bufs[b], gsems[b])
        if i >= 1:
            pb = (i - 1) % 2
            gd[pb].wait()
            _, poff, poh, pooff = chunks[i - 1]
            od[pb] = pltpu.async_copy(
                bufs[pb], poh.at[pl.ds(pooff, half)], osems[pb])
    last = len(chunks) - 1
    b = last % 2
    gd[b].wait()
    _, _, oh, ooff = chunks[last]
    od[b] = pltpu.async_copy(bufs[b], oh.at[pl.ds(ooff, half)], osems[b])
    od[(last - 1) % 2].wait()
    od[b].wait()


def _sc_gather(hid2, idx_s, idx_e):
    rows, d = hid2.shape
    bpw = rows // _NW
    mesh = plsc.VectorSubcoreMesh(core_axis_name="c", subcore_axis_name="s")
    f = pl.kernel(
        _sc_gather_body,
        out_type=(
            jax.ShapeDtypeStruct((rows, d), jnp.float32),
            jax.ShapeDtypeStruct((rows, d), jnp.float32),
        ),
        mesh=mesh,
        scratch_types=[
            pltpu.VMEM((bpw,), jnp.int32),
            pltpu.VMEM((bpw,), jnp.int32),
            pltpu.VMEM((bpw // 2, d), jnp.float32),
            pltpu.VMEM((bpw // 2, d), jnp.float32),
            pltpu.SemaphoreType.DMA,
            pltpu.SemaphoreType.DMA,
            pltpu.SemaphoreType.DMA,
            pltpu.SemaphoreType.DMA,
        ],
    )
    return f(hid2, idx_s, idx_e)


def _proj_body(x_ref, wp_ref, h_ref):
    h_ref[...] = lax.dot_general(
        x_ref[...], wp_ref[...], (((1,), (1,)), ((), ())),
        preferred_element_type=jnp.float32)


def _proj_call(x2, W_proj, bm):
    rows, d = x2.shape
    d3 = W_proj.shape[0]
    return pl.pallas_call(
        _proj_body,
        grid=(rows // bm,),
        in_specs=[
            pl.BlockSpec((bm, d), lambda i: (i, 0)),
            pl.BlockSpec((d3, d), lambda i: (0, 0)),
        ],
        out_specs=pl.BlockSpec((bm, d3), lambda i: (i, 0)),
        out_shape=jax.ShapeDtypeStruct((rows, d3), jnp.float32),
    )(x2, W_proj)


def _ffn_body(x_ref, m_ref, svs_ref, ip_ref, pe_ref, sve_ref, wp_ref,
              w1_ref, b1_ref, l1w_ref, l1b_ref, w2_ref, l2w_ref, l2b_ref,
              z_ref):
    x = x_ref[...]
    m = m_ref[...] > 0.0
    dd = x.shape[1]
    bm = x.shape[0]
    ffn = w1_ref.shape[0]
    pe_rows = pe_ref.shape[0]
    oh = (ip_ref[...] == lax.broadcasted_iota(
        jnp.int32, (bm, pe_rows), 1)).astype(jnp.float32)
    pe = lax.dot_general(oh, pe_ref[...], (((1,), (0,)), ((), ())),
                         preferred_element_type=jnp.float32)
    acc = jnp.zeros((bm, ffn), jnp.float32)
    svs = (svs_ref, pe, sve_ref)
    for c in range(3):
        wp_c = wp_ref[c * dd:(c + 1) * dd, :]
        h_c = lax.dot_general(x, wp_c, (((1,), (1,)), ((), ())),
                              preferred_element_type=jnp.float32)
        sv_c = svs[c] if c == 1 else svs[c][...]
        mc = jnp.where(m, sv_c, h_c)
        w1_c = w1_ref[:, c * dd:(c + 1) * dd]
        acc = acc + lax.dot_general(mc, w1_c, (((1,), (1,)), ((), ())),
                                    preferred_element_type=jnp.float32)
    t = _layer_norm(_gelu(acc + b1_ref[...]), l1w_ref[...], l1b_ref[...])
    z = lax.dot_general(t, w2_ref[...], (((1,), (1,)), ((), ())),
                        preferred_element_type=jnp.float32)
    z = _layer_norm(_gelu(z), l2w_ref[...], l2b_ref[...])
    z_ref[...] = z


def _ffn_call(x2, mask2, sv_s, idx_p2, pe_tab, sv_e, W_proj, W1, b1r,
              ln1_wr, ln1_br, W2, ln2_wr, ln2_br, bm):
    rows, d = x2.shape
    d3 = W_proj.shape[0]
    pe_rows = pe_tab.shape[0]
    ffn = W1.shape[0]
    g = rows // bm
    tok = lambda i: (i, 0)
    const = lambda i: (0, 0)
    return pl.pallas_call(
        _ffn_body,
        grid=(g,),
        in_specs=[
            pl.BlockSpec((bm, d), tok),
            pl.BlockSpec((bm, 1), tok),
            pl.BlockSpec((bm, d), tok),
            pl.BlockSpec((bm, 1), tok),
            pl.BlockSpec((pe_rows, d), const),
            pl.BlockSpec((bm, d), tok),
            pl.BlockSpec((d3, d), const),
            pl.BlockSpec((ffn, d3), const),
            pl.BlockSpec((1, ffn), const),
            pl.BlockSpec((1, ffn), const),
            pl.BlockSpec((1, ffn), const),
            pl.BlockSpec((d, ffn), const),
            pl.BlockSpec((1, d), const),
            pl.BlockSpec((1, d), const),
        ],
        out_specs=pl.BlockSpec((bm, d), tok),
        out_shape=jax.ShapeDtypeStruct((rows, d), jnp.float32),
    )(x2, mask2, sv_s, idx_p2, pe_tab, sv_e, W_proj, W1, b1r, ln1_wr,
      ln1_br, W2, ln2_wr, ln2_br)


def _cls_body(ss, bb, z_ref, wv_ref, bv_ref, out_ref):
    wv = wv_ref[...]
    bv = jnp.transpose(bv_ref[...], (1, 0))
    for b in range(bb):
        zb = z_ref[pl.ds(b * ss, ss), :]
        out_ref[:, b, :] = lax.dot_general(
            wv, zb, (((1,), (1,)), ((), ())),
            preferred_element_type=jnp.float32) + bv


def _cls_call(z2, W_cls, b_cls_c, vb, bb):
    rows, d = z2.shape
    ss = rows // bb
    v = W_cls.shape[0]
    nv = pl.cdiv(v, vb)
    return pl.pallas_call(
        functools.partial(_cls_body, ss, bb),
        grid=(nv,),
        in_specs=[
            pl.BlockSpec((rows, d), lambda j: (0, 0)),
            pl.BlockSpec((vb, d), lambda j: (j, 0)),
            pl.BlockSpec((1, vb), lambda j: (0, j)),
        ],
        out_specs=pl.BlockSpec((vb, bb, ss), lambda j: (j, 0, 0)),
        out_shape=jax.ShapeDtypeStruct((v, bb, ss), jnp.float32),
    )(z2, W_cls, b_cls_c)


def kernel(hidden_states, mask_labels, W_proj, span_pos_emb, W1, b1,
           ln1_w, ln1_b, W2, ln2_w, ln2_b, W_cls, b_cls):
    Bb, Ss, d = hidden_states.shape
    rows = Bb * Ss
    v = W_cls.shape[0]
    pe_rows = span_pos_emb.shape[0]

    idx_s, idx_p, idx_e, mask_f = _idx_call(mask_labels, pe_rows)
    hid2 = hidden_states.reshape(rows, d)
    sv_s, sv_e = _sc_gather(hid2, idx_s.reshape(rows), idx_e.reshape(rows))
    z = _ffn_call(
        hid2, mask_f.reshape(rows, 1), sv_s, idx_p.reshape(rows, 1),
        span_pos_emb, sv_e, W_proj, W1,
        b1.reshape(1, -1), ln1_w.reshape(1, -1), ln1_b.reshape(1, -1),
        W2, ln2_w.reshape(1, -1), ln2_b.reshape(1, -1), bm=256)
    logits_t = _cls_call(z, W_cls, b_cls.reshape(1, -1), vb=1024, bb=Bb)
    return jnp.transpose(logits_t, (1, 2, 0))


# speedup vs baseline: 1.5123x; 1.0385x over previous
import functools

import jax
import jax.numpy as jnp
from jax import lax
from jax.experimental import pallas as pl
from jax.experimental.pallas import tpu as pltpu
from jax.experimental.pallas import tpu_sc as plsc

_SC_CORES = 2
_SC_SUBCORES = 16
_NW = _SC_CORES * _SC_SUBCORES


def _erf(x):
    a1, a2, a3, a4, a5 = (0.254829592, -0.284496736, 1.421413741,
                          -1.453152027, 1.061405429)
    p = 0.3275911
    ax = jnp.abs(x)
    t = 1.0 / (1.0 + p * ax)
    poly = ((((a5 * t + a4) * t + a3) * t + a2) * t + a1) * t
    y = 1.0 - poly * jnp.exp(-ax * ax)
    return jnp.sign(x) * y


def _gelu(x):
    return 0.5 * x * (1.0 + _erf(x * 0.7071067811865476))


def _layer_norm(x, w, b, eps=1e-5):
    mu = jnp.mean(x, axis=-1, keepdims=True)
    var = jnp.mean((x - mu) ** 2, axis=-1, keepdims=True)
    return (x - mu) / jnp.sqrt(var + eps) * w + b


def _idx_body(pe_rows, ml_ref, idx_s_ref, idx_p_ref, idx_e_ref, mask_ref):
    ml = ml_ref[...]
    Bb, Ss = ml.shape
    mask = ml == 1
    j = lax.broadcasted_iota(jnp.int32, (Bb, Ss), 1)
    a = jnp.where(mask, jnp.int32(-1), j)
    d = 1
    while d < Ss:
        sh = jnp.concatenate(
            [jnp.full((Bb, d), -1, jnp.int32), a[:, : Ss - d]], axis=1)
        a = jnp.maximum(a, sh)
        d *= 2
    last_zero = a
    b = jnp.where(mask, jnp.int32(Ss), j)
    d = 1
    while d < Ss:
        sh = jnp.concatenate(
            [b[:, d:], jnp.full((Bb, d), Ss, jnp.int32)], axis=1)
        b = jnp.minimum(b, sh)
        d *= 2
    next_zero = b
    base = lax.broadcasted_iota(jnp.int32, (Bb, Ss), 0) * Ss
    i_s = jnp.where(last_zero < 0, Ss - 1, last_zero)
    i_e = jnp.minimum(next_zero, Ss - 1)
    i_p = jnp.clip(j - (last_zero + 1), 0, pe_rows - 1)
    idx_s_ref[...] = base + jnp.where(mask, i_s, j)
    idx_e_ref[...] = base + jnp.where(mask, i_e, j)
    idx_p_ref[...] = jnp.where(mask, i_p, 0)
    mask_ref[...] = mask.astype(jnp.float32)


def _idx_call(mask_labels, pe_rows):
    Bb, Ss = mask_labels.shape
    return pl.pallas_call(
        functools.partial(_idx_body, pe_rows),
        out_shape=(
            jax.ShapeDtypeStruct((Bb, Ss), jnp.int32),
            jax.ShapeDtypeStruct((Bb, Ss), jnp.int32),
            jax.ShapeDtypeStruct((Bb, Ss), jnp.int32),
            jax.ShapeDtypeStruct((Bb, Ss), jnp.float32),
        ),
    )(mask_labels)


def _sc_gather_body(hid, i_s, i_e, o_s, o_e,
                    idx_sv, idx_ev, buf_a, buf_b, gsa, gsb, osa, osb):
    wid = lax.axis_index("s") * _SC_CORES + lax.axis_index("c")
    bpw = o_s.shape[0] // _NW
    half = bpw // 2
    base = wid * bpw
    pltpu.sync_copy(i_s.at[pl.ds(base, bpw)], idx_sv)
    pltpu.sync_copy(i_e.at[pl.ds(base, bpw)], idx_ev)
    chunks = (
        (idx_sv, 0, o_s, base),
        (idx_sv, half, o_s, base + half),
        (idx_ev, 0, o_e, base),
        (idx_ev, half, o_e, base + half),
    )
    bufs = (buf_a, buf_b)
    gsems = (gsa, gsb)
    osems = (osa, osb)
    gd = [None, None]
    od = [None, None]
    for i, (iv, ioff, oh, ooff) in enumerate(chunks):
        b = i % 2
        if od[b] is not None:
            od[b].wait()
        gd[b] = pltpu.async_copy(
            hid.at[iv.at[pl.ds(ioff, half)]], bufs[b], gsems[b])
        if i >= 1:
            pb = (i - 1) % 2
            gd[pb].wait()
            _, poff, poh, pooff = chunks[i - 1]
            od[pb] = pltpu.async_copy(
                bufs[pb], poh.at[pl.ds(pooff, half)], osems[pb])
    last = len(chunks) - 1
    b = last % 2
    gd[b].wait()
    _, _, oh, ooff = chunks[last]
    od[b] = pltpu.async_copy(bufs[b], oh.at[pl.ds(ooff, half)], osems[b])
    od[(last - 1) % 2].wait()
    od[b].wait()


def _sc_gather(hid2, idx_s, idx_e):
    rows, d = hid2.shape
    bpw = rows // _NW
    mesh = plsc.VectorSubcoreMesh(core_axis_name="c", subcore_axis_name="s")
    f = pl.kernel(
        _sc_gather_body,
        out_type=(
            jax.ShapeDtypeStruct((rows, d), jnp.float32),
            jax.ShapeDtypeStruct((rows, d), jnp.float32),
        ),
        mesh=mesh,
        scratch_types=[
            pltpu.VMEM((bpw,), jnp.int32),
            pltpu.VMEM((bpw,), jnp.int32),
            pltpu.VMEM((bpw // 2, d), jnp.float32),
            pltpu.VMEM((bpw // 2, d), jnp.float32),
            pltpu.SemaphoreType.DMA,
            pltpu.SemaphoreType.DMA,
            pltpu.SemaphoreType.DMA,
            pltpu.SemaphoreType.DMA,
        ],
    )
    return f(hid2, idx_s, idx_e)


def _proj_body(x_ref, wp_ref, h_ref):
    h_ref[...] = lax.dot_general(
        x_ref[...], wp_ref[...], (((1,), (1,)), ((), ())),
        preferred_element_type=jnp.float32)


def _proj_call(x2, W_proj, bm):
    rows, d = x2.shape
    d3 = W_proj.shape[0]
    return pl.pallas_call(
        _proj_body,
        grid=(rows // bm,),
        in_specs=[
            pl.BlockSpec((bm, d), lambda i: (i, 0)),
            pl.BlockSpec((d3, d), lambda i: (0, 0)),
        ],
        out_specs=pl.BlockSpec((bm, d3), lambda i: (i, 0)),
        out_shape=jax.ShapeDtypeStruct((rows, d3), jnp.float32),
    )(x2, W_proj)


def _ffn_body(x_ref, m_ref, svs_ref, ip_ref, pe_ref, sve_ref, wp_ref,
              w1_ref, b1_ref, l1w_ref, l1b_ref, w2_ref, l2w_ref, l2b_ref,
              z_ref):
    x = x_ref[...]
    m = m_ref[...] > 0.0
    dd = x.shape[1]
    bm = x.shape[0]
    ffn = w1_ref.shape[0]
    pe_rows = pe_ref.shape[0]
    oh = (ip_ref[...] == lax.broadcasted_iota(
        jnp.int32, (bm, pe_rows), 1)).astype(jnp.float32)
    pe = lax.dot_general(oh, pe_ref[...], (((1,), (0,)), ((), ())),
                         preferred_element_type=jnp.float32)
    del ffn
    h_full = lax.dot_general(x, wp_ref[...], (((1,), (1,)), ((), ())),
                             preferred_element_type=jnp.float32)
    sv_full = jnp.concatenate([svs_ref[...], pe, sve_ref[...]], axis=1)
    mc = jnp.where(m, sv_full, h_full)
    acc = lax.dot_general(mc, w1_ref[...], (((1,), (1,)), ((), ())),
                          preferred_element_type=jnp.float32)
    t = _layer_norm(_gelu(acc + b1_ref[...]), l1w_ref[...], l1b_ref[...])
    z = lax.dot_general(t, w2_ref[...], (((1,), (1,)), ((), ())),
                        preferred_element_type=jnp.float32)
    z = _layer_norm(_gelu(z), l2w_ref[...], l2b_ref[...])
    z_ref[...] = z


def _ffn_call(x2, mask2, sv_s, idx_p2, pe_tab, sv_e, W_proj, W1, b1r,
              ln1_wr, ln1_br, W2, ln2_wr, ln2_br, bm):
    rows, d = x2.shape
    d3 = W_proj.shape[0]
    pe_rows = pe_tab.shape[0]
    ffn = W1.shape[0]
    g = rows // bm
    tok = lambda i: (i, 0)
    const = lambda i: (0, 0)
    return pl.pallas_call(
        _ffn_body,
        grid=(g,),
        in_specs=[
            pl.BlockSpec((bm, d), tok),
            pl.BlockSpec((bm, 1), tok),
            pl.BlockSpec((bm, d), tok),
            pl.BlockSpec((bm, 1), tok),
            pl.BlockSpec((pe_rows, d), const),
            pl.BlockSpec((bm, d), tok),
            pl.BlockSpec((d3, d), const),
            pl.BlockSpec((ffn, d3), const),
            pl.BlockSpec((1, ffn), const),
            pl.BlockSpec((1, ffn), const),
            pl.BlockSpec((1, ffn), const),
            pl.BlockSpec((d, ffn), const),
            pl.BlockSpec((1, d), const),
            pl.BlockSpec((1, d), const),
        ],
        out_specs=pl.BlockSpec((bm, d), tok),
        out_shape=jax.ShapeDtypeStruct((rows, d), jnp.float32),
    )(x2, mask2, sv_s, idx_p2, pe_tab, sv_e, W_proj, W1, b1r, ln1_wr,
      ln1_br, W2, ln2_wr, ln2_br)


def _cls_body(ss, bb, z_ref, wv_ref, bv_ref, out_ref):
    wv = wv_ref[...]
    bv = jnp.transpose(bv_ref[...], (1, 0))
    for b in range(bb):
        zb = z_ref[pl.ds(b * ss, ss), :]
        out_ref[:, b, :] = lax.dot_general(
            wv, zb, (((1,), (1,)), ((), ())),
            preferred_element_type=jnp.float32) + bv


def _cls_call(z2, W_cls, b_cls_c, vb, bb):
    rows, d = z2.shape
    ss = rows // bb
    v = W_cls.shape[0]
    nv = pl.cdiv(v, vb)
    return pl.pallas_call(
        functools.partial(_cls_body, ss, bb),
        grid=(nv,),
        in_specs=[
            pl.BlockSpec((rows, d), lambda j: (0, 0)),
            pl.BlockSpec((vb, d), lambda j: (j, 0)),
            pl.BlockSpec((1, vb), lambda j: (0, j)),
        ],
        out_specs=pl.BlockSpec((vb, bb, ss), lambda j: (j, 0, 0)),
        out_shape=jax.ShapeDtypeStruct((v, bb, ss), jnp.float32),
    )(z2, W_cls, b_cls_c)


def kernel(hidden_states, mask_labels, W_proj, span_pos_emb, W1, b1,
           ln1_w, ln1_b, W2, ln2_w, ln2_b, W_cls, b_cls):
    Bb, Ss, d = hidden_states.shape
    rows = Bb * Ss
    v = W_cls.shape[0]
    pe_rows = span_pos_emb.shape[0]

    idx_s, idx_p, idx_e, mask_f = _idx_call(mask_labels, pe_rows)
    hid2 = hidden_states.reshape(rows, d)
    sv_s, sv_e = _sc_gather(hid2, idx_s.reshape(rows), idx_e.reshape(rows))
    z = _ffn_call(
        hid2, mask_f.reshape(rows, 1), sv_s, idx_p.reshape(rows, 1),
        span_pos_emb, sv_e, W_proj, W1,
        b1.reshape(1, -1), ln1_w.reshape(1, -1), ln1_b.reshape(1, -1),
        W2, ln2_w.reshape(1, -1), ln2_b.reshape(1, -1), bm=256)
    logits_t = _cls_call(z, W_cls, b_cls.reshape(1, -1), vb=1024, bb=Bb)
    return jnp.transpose(logits_t, (1, 2, 0))
